# baseline (speedup 1.0000x reference)
import functools

import jax
import jax.numpy as jnp
from jax import lax
from jax.experimental import pallas as pl
from jax.experimental.pallas import tpu as pltpu
from jax.experimental.pallas import tpu_sc as plsc

N = 10000
D = 256
E = 160000
HALF = 128
NCORE = 2
NSUB = 16
CHUNK = 128
NB = 80
E_PAD = NSUB * NB * CHUNK
ROWS_ACC = N + 16
RPW = 624
TAIL = N - NSUB * RPW


def _sc_aggregate(x, packed):
    mesh = plsc.VectorSubcoreMesh(core_axis_name="c", subcore_axis_name="s")

    @functools.partial(
        pl.kernel,
        mesh=mesh,
        out_type=jax.ShapeDtypeStruct((NCORE, N, HALF), jnp.float32),
        scratch_types=[
            pltpu.VMEM((NB, CHUNK), jnp.int32),
            pltpu.VMEM((8, CHUNK), jnp.int32),
            pltpu.VMEM((2, CHUNK, HALF), jnp.float32),
            pltpu.VMEM_SHARED((ROWS_ACC, HALF), jnp.float32),
            pltpu.SemaphoreType.DMA,
            pltpu.SemaphoreType.DMA,
        ],
    )
    def k(packed_hbm, x_hbm, out_hbm, pk_v, su, rows_v, acc, sem0, sem1):
        c = lax.axis_index("c")
        s = lax.axis_index("s")
        xh = x_hbm.at[:, pl.ds(c * HALF, HALF)]

        pltpu.sync_copy(packed_hbm.at[s], pk_v)

        def zrow(i, carry):
            def zcol(j, carry2):
                rows_v[0, i, pl.ds(j * 16, 16)] = jnp.zeros((16,), jnp.float32)
                return carry2
            return lax.fori_loop(0, HALF // 16, zcol, carry)
        lax.fori_loop(0, CHUNK, zrow, 0)
        zslab = rows_v.at[0]
        for t in range(RPW // CHUNK):
            pltpu.sync_copy(zslab, acc.at[pl.ds(s * RPW + t * CHUNK, CHUNK), :])
        rem = RPW - (RPW // CHUNK) * CHUNK
        if rem:
            pltpu.sync_copy(zslab.at[pl.ds(0, rem), :],
                            acc.at[pl.ds(s * RPW + RPW - rem, rem), :])

        @pl.when(s == NSUB - 1)
        def _zero_tail():
            pltpu.sync_copy(zslab.at[pl.ds(0, TAIL), :],
                            acc.at[pl.ds(NSUB * RPW, TAIL), :])
        plsc.subcore_barrier()

        sems = (sem0, sem1)

        def unpack(j, buf):
            for t in range(CHUNK // 16):
                p = pk_v[j, pl.ds(t * 16, 16)]
                su[buf, pl.ds(t * 16, 16)] = jnp.right_shift(p, 14)
                su[2 + buf, pl.ds(t * 16, 16)] = jnp.bitwise_and(p, 16383)

        def start_gather(j, buf):
            unpack(j, buf)
            pltpu.async_copy(xh.at[su.at[buf]], rows_v.at[buf], sems[buf])

        def finish_chunk(buf):
            pltpu.make_async_copy(xh.at[su.at[buf]], rows_v.at[buf],
                                  sems[buf]).wait()
            pltpu.sync_copy(rows_v.at[buf], acc.at[su.at[2 + buf]], add=True)

        start_gather(0, 0)

        def body(t, carry):
            j = 2 * t
            for bb in range(2):
                jj = j + bb

                @pl.when(jj + 1 < NB)
                def _next():
                    start_gather(jj + 1, 1 - bb)

                finish_chunk(bb)
            return carry
        lax.fori_loop(0, NB // 2, body, 0)
        plsc.subcore_barrier()

        pltpu.sync_copy(acc.at[pl.ds(s * RPW, RPW), :],
                        out_hbm.at[c, pl.ds(s * RPW, RPW)])

        @pl.when(s == NSUB - 1)
        def _write_tail():
            pltpu.sync_copy(acc.at[pl.ds(NSUB * RPW, TAIL), :],
                            out_hbm.at[c, pl.ds(NSUB * RPW, TAIL)])

    return k(packed, x)


def _tc_pre(x, W, b, eps):
    R = 1000

    def body(eps_ref, x_ref, w_ref, b_ref, o_ref):
        e1 = 1.0 + eps_ref[0, 0]
        acc = lax.dot_general(x_ref[...], w_ref[...], (((1,), (1,)), ((), ())),
                              preferred_element_type=jnp.float32)
        o_ref[...] = e1 * acc + b_ref[...]

    return pl.pallas_call(
        body,
        grid=(N // R,),
        in_specs=[
            pl.BlockSpec(memory_space=pltpu.SMEM),
            pl.BlockSpec((R, D), lambda i: (i, 0)),
            pl.BlockSpec((D, D), lambda i: (0, 0)),
            pl.BlockSpec((1, D), lambda i: (0, 0)),
        ],
        out_specs=pl.BlockSpec((R, D), lambda i: (i, 0)),
        out_shape=jax.ShapeDtypeStruct((N, D), jnp.float32),
    )(eps.reshape(1, 1).astype(jnp.float32), x, W, b.reshape(1, D))


def _tc_post(P, aggr2, W):
    R = 1000

    def body(p_ref, a_ref, w_ref, o_ref):
        w = w_ref[...]
        acc = lax.dot_general(a_ref[0], w[:, :HALF], (((1,), (1,)), ((), ())),
                              preferred_element_type=jnp.float32)
        acc = acc + lax.dot_general(a_ref[1], w[:, HALF:], (((1,), (1,)), ((), ())),
                                    preferred_element_type=jnp.float32)
        o_ref[...] = jnp.maximum(acc + p_ref[...], 0.0)

    return pl.pallas_call(
        body,
        grid=(N // R,),
        in_specs=[
            pl.BlockSpec((R, D), lambda i: (i, 0)),
            pl.BlockSpec((NCORE, R, HALF), lambda i: (0, i, 0)),
            pl.BlockSpec((D, D), lambda i: (0, 0)),
        ],
        out_specs=pl.BlockSpec((R, D), lambda i: (i, 0)),
        out_shape=jax.ShapeDtypeStruct((N, D), jnp.float32),
    )(P, aggr2, W)


def kernel(x, edge_index, W, b, eps):
    src = edge_index[0]
    dst = edge_index[1]
    pad = E_PAD - E
    pad_src = jnp.arange(pad, dtype=jnp.int32) % jnp.int32(N)
    pad_dst = jnp.int32(N) + jnp.arange(pad, dtype=jnp.int32) % jnp.int32(16)
    srcp = jnp.concatenate([src, pad_src])
    dstp = jnp.concatenate([dst, pad_dst])
    packed = (srcp * 16384 + dstp).reshape(NSUB, NB, CHUNK)
    aggr2 = _sc_aggregate(x, packed)
    P = _tc_pre(x, W, b, eps)
    return _tc_post(P, aggr2, W)


# speedup vs baseline: 1.0246x; 1.0246x over previous
import functools

import jax
import jax.numpy as jnp
from jax import lax
from jax.experimental import pallas as pl
from jax.experimental.pallas import tpu as pltpu
from jax.experimental.pallas import tpu_sc as plsc

N = 10000
D = 256
E = 160000
HALF = 128
NCORE = 2
NSUB = 16
CHUNK = 128
NB = 80
E_PAD = NSUB * NB * CHUNK
NRING = 3
ROWS_ACC = N + 16
RPW = 624
TAIL = N - NSUB * RPW


def _sc_aggregate(x, packed):
    mesh = plsc.VectorSubcoreMesh(core_axis_name="c", subcore_axis_name="s")

    @functools.partial(
        pl.kernel,
        mesh=mesh,
        out_type=jax.ShapeDtypeStruct((NCORE, N, HALF), jnp.float32),
        scratch_types=[
            pltpu.VMEM((8, CHUNK), jnp.int32),
            pltpu.VMEM((NRING, CHUNK, HALF), jnp.float32),
            pltpu.VMEM_SHARED((ROWS_ACC, HALF), jnp.float32),
        ] + [pltpu.SemaphoreType.DMA] * (2 * NRING),
    )
    def k(packed_hbm, x_hbm, out_hbm, su, rows_v, acc, *sems):
        c = lax.axis_index("c")
        s = lax.axis_index("s")
        xh = x_hbm.at[:, pl.ds(c * HALF, HALF)]
        pksems = sems[:NRING]
        gsems = sems[NRING:]

        def zrow(i, carry):
            def zcol(j, carry2):
                rows_v[0, i, pl.ds(j * 16, 16)] = jnp.zeros((16,), jnp.float32)
                return carry2
            return lax.fori_loop(0, HALF // 16, zcol, carry)
        lax.fori_loop(0, CHUNK, zrow, 0)
        zslab = rows_v.at[0]
        for t in range(RPW // CHUNK):
            pltpu.sync_copy(zslab, acc.at[pl.ds(s * RPW + t * CHUNK, CHUNK), :])
        rem = RPW - (RPW // CHUNK) * CHUNK
        if rem:
            pltpu.sync_copy(zslab.at[pl.ds(0, rem), :],
                            acc.at[pl.ds(s * RPW + RPW - rem, rem), :])

        @pl.when(s == NSUB - 1)
        def _zero_tail():
            pltpu.sync_copy(zslab.at[pl.ds(0, TAIL), :],
                            acc.at[pl.ds(NSUB * RPW, TAIL), :])
        plsc.subcore_barrier()

        def start_pk(j, r):
            pltpu.async_copy(packed_hbm.at[s, j], su.at[r], pksems[r])

        def start_gather(j, r):
            pltpu.make_async_copy(packed_hbm.at[s, j], su.at[r],
                                  pksems[r]).wait()
            for t in range(CHUNK // 16):
                p = su[r, pl.ds(t * 16, 16)]
                su[r, pl.ds(t * 16, 16)] = jnp.right_shift(p, 14)
                su[4 + r, pl.ds(t * 16, 16)] = jnp.bitwise_and(p, 16383)
            pltpu.async_copy(xh.at[su.at[r]], rows_v.at[r], gsems[r])

        def finish_chunk(r):
            pltpu.make_async_copy(xh.at[su.at[r]], rows_v.at[r],
                                  gsems[r]).wait()
            pltpu.sync_copy(rows_v.at[r], acc.at[su.at[4 + r]], add=True)

        for j in range(NRING):
            start_pk(j, j)
        for j in range(NRING - 1):
            start_gather(j, j)

        def body(t, carry):
            j = NRING * t
            for bb in range(NRING):
                jj = j + bb
                r = bb

                @pl.when(jj < NB)
                def _this():
                    finish_chunk(r)

                @pl.when(jj + NRING < NB)
                def _pk():
                    start_pk(jj + NRING, r)

                @pl.when(jj + NRING - 1 < NB)
                def _gather():
                    start_gather(jj + NRING - 1, (r + NRING - 1) % NRING)
            return carry
        lax.fori_loop(0, (NB + NRING - 1) // NRING, body, 0)
        plsc.subcore_barrier()

        pltpu.sync_copy(acc.at[pl.ds(s * RPW, RPW), :],
                        out_hbm.at[c, pl.ds(s * RPW, RPW)])

        @pl.when(s == NSUB - 1)
        def _write_tail():
            pltpu.sync_copy(acc.at[pl.ds(NSUB * RPW, TAIL), :],
                            out_hbm.at[c, pl.ds(NSUB * RPW, TAIL)])

    return k(packed, x)


def _tc_dense(x, aggr2, W, b, eps):
    R = 1000

    def body(eps_ref, x_ref, a_ref, w_ref, b_ref, o_ref):
        e1 = 1.0 + eps_ref[0, 0]
        w = w_ref[...]
        h0 = e1 * x_ref[:, :HALF] + a_ref[0]
        h1 = e1 * x_ref[:, HALF:] + a_ref[1]
        acc = lax.dot_general(h0, w[:, :HALF], (((1,), (1,)), ((), ())),
                              preferred_element_type=jnp.float32)
        acc = acc + lax.dot_general(h1, w[:, HALF:], (((1,), (1,)), ((), ())),
                                    preferred_element_type=jnp.float32)
        o_ref[...] = jnp.maximum(acc + b_ref[...], 0.0)

    return pl.pallas_call(
        body,
        grid=(N // R,),
        in_specs=[
            pl.BlockSpec(memory_space=pltpu.SMEM),
            pl.BlockSpec((R, D), lambda i: (i, 0)),
            pl.BlockSpec((NCORE, R, HALF), lambda i: (0, i, 0)),
            pl.BlockSpec((D, D), lambda i: (0, 0)),
            pl.BlockSpec((1, D), lambda i: (0, 0)),
        ],
        out_specs=pl.BlockSpec((R, D), lambda i: (i, 0)),
        out_shape=jax.ShapeDtypeStruct((N, D), jnp.float32),
    )(eps.reshape(1, 1).astype(jnp.float32), x, aggr2, W, b.reshape(1, D))


def kernel(x, edge_index, W, b, eps):
    src = edge_index[0]
    dst = edge_index[1]
    pad = E_PAD - E
    pad_src = jnp.arange(pad, dtype=jnp.int32) % jnp.int32(N)
    pad_dst = jnp.int32(N) + jnp.arange(pad, dtype=jnp.int32) % jnp.int32(16)
    srcp = jnp.concatenate([src, pad_src])
    dstp = jnp.concatenate([dst, pad_dst])
    packed = (srcp * 16384 + dstp).reshape(NSUB, NB, CHUNK)
    aggr2 = _sc_aggregate(x, packed)
    return _tc_dense(x, aggr2, W, b, eps)
